# Initial kernel scaffold; baseline (speedup 1.0000x reference)
import jax
import jax.numpy as jnp
from jax.experimental import pallas as pl


def kernel(inputs, embedding_table):
    raise NotImplementedError("write your pallas kernel here")


# speedup vs baseline: 2.9631x; 2.9631x over previous
import functools

import jax
import jax.numpy as jnp
from jax import lax
from jax.experimental import pallas as pl
from jax.experimental.pallas import tpu as pltpu
from jax.experimental.pallas import tpu_sc as plsc

_VOCAB = 100000
_D = 128
_B = 4096
_S = 50
_N = _B * _S
_NC, _NS = 2, 16
_NW = _NC * _NS
_ROWS_PER_W = _N // _NW
_CHUNK = 128
_NCHUNKS = _ROWS_PER_W // _CHUNK


def _build_lookup():
    mesh = plsc.VectorSubcoreMesh(core_axis_name="c", subcore_axis_name="s")

    @functools.partial(
        pl.kernel,
        mesh=mesh,
        out_type=jax.ShapeDtypeStruct((_N, _D), jnp.float32),
        scratch_types=[
            pltpu.VMEM((_NCHUNKS, _CHUNK), jnp.int32),
            pltpu.VMEM((_CHUNK, _D), jnp.float32),
            pltpu.SemaphoreType.DMA,
        ],
    )
    def lookup(idx_hbm, table_hbm, out_hbm, idx_v, rows_v, sem_g):
        wid = lax.axis_index("s") * _NC + lax.axis_index("c")
        base = wid * _ROWS_PER_W
        pltpu.sync_copy(idx_hbm.at[wid], idx_v)

        def body(j, carry):
            pltpu.async_copy(table_hbm.at[idx_v.at[j]], rows_v, sem_g).wait()
            pltpu.sync_copy(rows_v, out_hbm.at[pl.ds(base + j * _CHUNK, _CHUNK)])
            return carry

        lax.fori_loop(0, _NCHUNKS, body, 0)

    return lookup


_lookup = _build_lookup()


def kernel(inputs, embedding_table):
    idx = inputs.reshape(_NW, _NCHUNKS, _CHUNK)
    out = _lookup(idx, embedding_table)
    return out.reshape(_B, _S, _D)


# speedup vs baseline: 3.3398x; 1.1271x over previous
import functools

import jax
import jax.numpy as jnp
from jax import lax
from jax.experimental import pallas as pl
from jax.experimental.pallas import tpu as pltpu
from jax.experimental.pallas import tpu_sc as plsc

_VOCAB = 100000
_D = 128
_B = 4096
_S = 50
_N = _B * _S
_NC, _NS = 2, 16
_NW = _NC * _NS
_ROWS_PER_W = _N // _NW
_CHUNK = 128
_NCHUNKS = _ROWS_PER_W // _CHUNK
_NBUF = 5
_LEAD = 3
_NGRP = _NCHUNKS // _NBUF


def _build_lookup():
    mesh = plsc.VectorSubcoreMesh(core_axis_name="c", subcore_axis_name="s")

    scratch = [
        pltpu.VMEM((_NCHUNKS, _CHUNK), jnp.int32),
        pltpu.VMEM((_NBUF, _CHUNK, _D), jnp.float32),
    ] + [pltpu.SemaphoreType.DMA] * (2 * _NBUF)

    @functools.partial(
        pl.kernel,
        mesh=mesh,
        out_type=jax.ShapeDtypeStruct((_N, _D), jnp.float32),
        scratch_types=scratch,
    )
    def lookup(idx_hbm, table_hbm, out_hbm, idx_v, rows, *sems):
        sem_g = sems[:_NBUF]
        sem_s = sems[_NBUF:]
        wid = lax.axis_index("s") * _NC + lax.axis_index("c")
        base = wid * _ROWS_PER_W
        pltpu.sync_copy(idx_hbm.at[wid], idx_v)

        def gather(j, b):
            pltpu.async_copy(table_hbm.at[idx_v.at[j]], rows.at[b], sem_g[b])

        def gather_wait(b):
            pltpu.make_async_copy(
                table_hbm.at[idx_v.at[0]], rows.at[b], sem_g[b]
            ).wait()

        def store(k, b):
            pltpu.async_copy(
                rows.at[b], out_hbm.at[pl.ds(base + k * _CHUNK, _CHUNK)], sem_s[b]
            )

        def store_wait(b):
            pltpu.make_async_copy(
                rows.at[b], out_hbm.at[pl.ds(0, _CHUNK)], sem_s[b]
            ).wait()

        for j in range(_LEAD):
            gather(j, j)

        def step(k, b, first_group):
            fb = (b + _LEAD) % _NBUF
            if not (first_group and b < _NBUF - _LEAD):
                store_wait(fb)
            gather(k + _LEAD, fb)
            gather_wait(b)
            store(k, b)

        for b in range(_NBUF):
            step(b, b, True)

        def body(g, carry):
            k0 = g * _NBUF
            for b in range(_NBUF):
                step(k0 + b, b, False)
            return carry

        lax.fori_loop(1, _NGRP - 1, body, 0)

        k0 = (_NGRP - 1) * _NBUF
        for b in range(_NBUF):
            k = k0 + b
            f = k + _LEAD
            if f < _NCHUNKS:
                fb = (b + _LEAD) % _NBUF
                store_wait(fb)
                gather(f, fb)
            gather_wait(b)
            store(k, b)

        for b in range(_NBUF):
            store_wait(b)

    return lookup


_lookup = _build_lookup()


def kernel(inputs, embedding_table):
    idx = inputs.reshape(_NW, _NCHUNKS, _CHUNK)
    out = _lookup(idx, embedding_table)
    return out.reshape(_B, _S, _D)


# speedup vs baseline: 5.7348x; 1.7171x over previous
import functools

import jax
import jax.numpy as jnp
from jax import lax
from jax.experimental import pallas as pl
from jax.experimental.pallas import tpu as pltpu
from jax.experimental.pallas import tpu_sc as plsc

_VOCAB = 100000
_D = 128
_B = 4096
_S = 50
_NC, _NS = 2, 16
_NW = _NC * _NS
_BPW = _B // _NW
_G = 4
_NGROUPS = _BPW // _G
_NBUF = 4
_LEAD = 2


def _build_lookup():
    mesh = plsc.VectorSubcoreMesh(core_axis_name="c", subcore_axis_name="s")

    scratch = [
        pltpu.VMEM((_BPW, _S), jnp.int32),
        pltpu.VMEM((_NBUF, _G, _S, _D), jnp.float32),
    ] + [pltpu.SemaphoreType.DMA] * (2 * _NBUF)

    @functools.partial(
        pl.kernel,
        mesh=mesh,
        out_type=jax.ShapeDtypeStruct((_B, _S, _D), jnp.float32),
        scratch_types=scratch,
    )
    def lookup(idx_hbm, table_hbm, out_hbm, idx_v, rows, *sems):
        sem_g = sems[:_NBUF]
        sem_s = sems[_NBUF:]
        wid = lax.axis_index("s") * _NC + lax.axis_index("c")
        base = wid * _BPW
        pltpu.sync_copy(idx_hbm.at[wid], idx_v)

        def gather_group(g, b):
            for i in range(_G):
                pltpu.async_copy(
                    table_hbm.at[idx_v.at[g * _G + i]], rows.at[b].at[i], sem_g[b]
                )

        def gather_wait(b):
            for i in range(_G):
                pltpu.make_async_copy(
                    table_hbm.at[idx_v.at[0]], rows.at[b].at[i], sem_g[b]
                ).wait()

        def store(g, b):
            pltpu.async_copy(
                rows.at[b], out_hbm.at[pl.ds(base + g * _G, _G)], sem_s[b]
            )

        def store_wait(b):
            pltpu.make_async_copy(
                rows.at[b], out_hbm.at[pl.ds(0, _G)], sem_s[b]
            ).wait()

        for g in range(_LEAD):
            gather_group(g, g)

        for g in range(_NGROUPS):
            b = g % _NBUF
            if g + _LEAD < _NGROUPS:
                fb = (g + _LEAD) % _NBUF
                if g >= _NBUF - _LEAD:
                    store_wait(fb)
                gather_group(g + _LEAD, fb)
            gather_wait(b)
            store(g, b)

        for b in range(_NBUF):
            store_wait(b)

    return lookup


_lookup = _build_lookup()


def kernel(inputs, embedding_table):
    idx = inputs.reshape(_NW, _BPW, _S)
    return _lookup(idx, embedding_table)
